# Initial kernel scaffold; baseline (speedup 1.0000x reference)
import functools

import jax
import jax.numpy as jnp
from jax import lax
from jax.experimental import pallas as pl
from jax.experimental.pallas import tpu as pltpu
from jax.experimental.pallas import tpu_sc as plsc

N = 10000
E = 320000
H = 128
C = 40

NC = 2
NS = 16
NW = NC * NS
EPW = E // NW
CS = 100
NCHUNK = EPW // CS
RPT = N // NS


def _sc_mesh():
    return plsc.VectorSubcoreMesh(core_axis_name="c", subcore_axis_name="s")


def _deg_body(dst_hbm, ones_hbm, zeros_hbm, out_hbm, deg_sh, dst_v, ones_v):
    cid = lax.axis_index("c")
    sid = lax.axis_index("s")
    wid = cid * NS + sid
    row0 = sid * RPT
    pltpu.sync_copy(zeros_hbm, deg_sh.at[pl.ds(row0, RPT)])
    pltpu.sync_copy(ones_hbm, ones_v)
    pltpu.sync_copy(dst_hbm.at[wid], dst_v)
    plsc.subcore_barrier()

    def chunk(j, _):
        pltpu.sync_copy(ones_v, deg_sh.at[dst_v.at[j]], add=True)
        return _

    lax.fori_loop(0, NCHUNK, chunk, None)
    plsc.subcore_barrier()
    pltpu.sync_copy(deg_sh.at[pl.ds(row0, RPT)], out_hbm.at[cid, pl.ds(row0, RPT)])


def _sc_degree(dst_r, ones16, zeros16):
    f = pl.kernel(
        _deg_body,
        out_type=jax.ShapeDtypeStruct((NC, N, 16), jnp.float32),
        mesh=_sc_mesh(),
        scratch_types=[
            pltpu.VMEM_SHARED((N, 16), jnp.float32),
            pltpu.VMEM((NCHUNK, CS), jnp.int32),
            pltpu.VMEM((CS, 16), jnp.float32),
        ],
    )
    return f(dst_r, ones16, zeros16)


def _acc_body(h_hbm, src_hbm, dst_hbm, zeros_hbm, out_hbm,
              acc_sh, src_v, dst_v, rows_v, sem):
    cid = lax.axis_index("c")
    sid = lax.axis_index("s")
    wid = cid * NS + sid
    row0 = sid * RPT
    pltpu.sync_copy(zeros_hbm, acc_sh.at[pl.ds(row0, RPT)])
    pltpu.sync_copy(src_hbm.at[wid], src_v)
    pltpu.sync_copy(dst_hbm.at[wid], dst_v)
    plsc.subcore_barrier()

    def chunk(j, _):
        pltpu.async_copy(h_hbm.at[src_v.at[j]], rows_v, sem).wait()
        pltpu.sync_copy(rows_v, acc_sh.at[dst_v.at[j]], add=True)
        return _

    lax.fori_loop(0, NCHUNK, chunk, None)
    plsc.subcore_barrier()
    pltpu.sync_copy(acc_sh.at[pl.ds(row0, RPT)], out_hbm.at[cid, pl.ds(row0, RPT)])


def _sc_accumulate(h_s, src_r, dst_r, zeros128):
    f = pl.kernel(
        _acc_body,
        out_type=jax.ShapeDtypeStruct((NC, N, H), jnp.float32),
        mesh=_sc_mesh(),
        scratch_types=[
            pltpu.VMEM_SHARED((N, H), jnp.float32),
            pltpu.VMEM((NCHUNK, CS), jnp.int32),
            pltpu.VMEM((NCHUNK, CS), jnp.int32),
            pltpu.VMEM((CS, H), jnp.float32),
            pltpu.SemaphoreType.DMA,
        ],
    )
    return f(h_s, src_r, dst_r, zeros128)


R = 500


def _dinv(deg0, deg1):
    return lax.rsqrt(deg0[:, 0:1] + deg1[:, 0:1] + 1.0)


def _first_body(x_ref, w_ref, deg0_ref, deg1_ref, o_ref):
    dinv = _dinv(deg0_ref[...], deg1_ref[...])
    h = jnp.dot(x_ref[...], w_ref[...], preferred_element_type=jnp.float32)
    o_ref[...] = h * dinv


def _mid_body(a0_ref, a1_ref, hs_ref, deg0_ref, deg1_ref, b_ref, w_ref, o_ref):
    dinv = _dinv(deg0_ref[...], deg1_ref[...])
    z = a0_ref[...] + a1_ref[...] + hs_ref[...]
    z = jnp.maximum(z * dinv + b_ref[...], 0.0)
    o_ref[...] = jnp.dot(z, w_ref[...], preferred_element_type=jnp.float32) * dinv


def _head_body(a0_ref, a1_ref, hs_ref, deg0_ref, deg1_ref, b3_ref,
               wl1_ref, bl1_ref, wl2_ref, bl2_ref, o_ref):
    dinv = _dinv(deg0_ref[...], deg1_ref[...])
    z = a0_ref[...] + a1_ref[...] + hs_ref[...]
    z = jnp.maximum(z * dinv + b3_ref[...], 0.0)
    h4 = jnp.dot(z, wl1_ref[...], preferred_element_type=jnp.float32)
    h4 = jnp.maximum(h4 + bl1_ref[...], 0.0)
    logits = jnp.dot(h4, wl2_ref[...], preferred_element_type=jnp.float32)
    logits = logits + bl2_ref[...]
    col = lax.broadcasted_iota(jnp.int32, logits.shape, 1)
    logits = jnp.where(col < C, logits, -1e30)
    m = jnp.max(logits, axis=-1, keepdims=True)
    lse = jnp.log(jnp.sum(jnp.exp(logits - m), axis=-1, keepdims=True))
    o_ref[...] = logits - m - lse


def _row_spec(width):
    return pl.BlockSpec((R, width), lambda i: (i, 0))


def _full_spec(shape):
    return pl.BlockSpec(shape, lambda i: (0,) * len(shape))


def _tc_first(x, w, deg0, deg1):
    return pl.pallas_call(
        _first_body,
        grid=(N // R,),
        in_specs=[_row_spec(H), _full_spec((H, H)), _row_spec(16), _row_spec(16)],
        out_specs=_row_spec(H),
        out_shape=jax.ShapeDtypeStruct((N, H), jnp.float32),
    )(x, w, deg0, deg1)


def _tc_mid(a0, a1, hs, deg0, deg1, b, w):
    return pl.pallas_call(
        _mid_body,
        grid=(N // R,),
        in_specs=[_row_spec(H), _row_spec(H), _row_spec(H), _row_spec(16),
                  _row_spec(16), _full_spec((1, H)), _full_spec((H, H))],
        out_specs=_row_spec(H),
        out_shape=jax.ShapeDtypeStruct((N, H), jnp.float32),
    )(a0, a1, hs, deg0, deg1, b, w)


def _tc_head(a0, a1, hs, deg0, deg1, b3, wl1, bl1, wl2p, bl2p):
    return pl.pallas_call(
        _head_body,
        grid=(N // R,),
        in_specs=[_row_spec(H), _row_spec(H), _row_spec(H), _row_spec(16),
                  _row_spec(16), _full_spec((1, H)), _full_spec((H, H)),
                  _full_spec((1, H)), _full_spec((H, H)), _full_spec((1, H))],
        out_specs=_row_spec(H),
        out_shape=jax.ShapeDtypeStruct((N, H), jnp.float32),
    )(a0, a1, hs, deg0, deg1, b3, wl1, bl1, wl2p, bl2p)


def kernel(x, edge_index, batch, W1, b1, W2, b2, W3, b3, Wl1, bl1, Wl2, bl2):
    del batch
    src_r = edge_index[0].reshape(NW, NCHUNK, CS)
    dst_r = edge_index[1].reshape(NW, NCHUNK, CS)
    ones16 = jnp.ones((CS, 16), jnp.float32)
    zeros16 = jnp.zeros((RPT, 16), jnp.float32)
    zeros128 = jnp.zeros((RPT, H), jnp.float32)

    deg = _sc_degree(dst_r, ones16, zeros16)
    deg0, deg1 = deg[0], deg[1]

    hs = _tc_first(x, W1, deg0, deg1)
    acc = _sc_accumulate(hs, src_r, dst_r, zeros128)
    hs = _tc_mid(acc[0], acc[1], hs, deg0, deg1, b1.reshape(1, H), W2)
    acc = _sc_accumulate(hs, src_r, dst_r, zeros128)
    hs = _tc_mid(acc[0], acc[1], hs, deg0, deg1, b2.reshape(1, H), W3)
    acc = _sc_accumulate(hs, src_r, dst_r, zeros128)

    wl2p = jnp.pad(Wl2, ((0, 0), (0, H - C)))
    bl2p = jnp.pad(bl2, (0, H - C)).reshape(1, H)
    out = _tc_head(acc[0], acc[1], hs, deg0, deg1, b3.reshape(1, H),
                   Wl1, bl1.reshape(1, H), wl2p, bl2p)
    return out[:, :C]


# speedup vs baseline: 15.2525x; 15.2525x over previous
import functools

import jax
import jax.numpy as jnp
from jax import lax
from jax.experimental import pallas as pl
from jax.experimental.pallas import tpu as pltpu
from jax.experimental.pallas import tpu_sc as plsc

N = 10000
E = 320000
H = 128
C = 40

NC = 2
NS = 16
NW = NC * NS
EPW = E // NW
CS = 100
NCHUNK = EPW // CS
NP = 10240
RPT = NP // NS


def _sc_mesh():
    return plsc.VectorSubcoreMesh(core_axis_name="c", subcore_axis_name="s")


def _deg_body(dst_hbm, ones_hbm, zeros_hbm, out_hbm, deg_sh, dst_v, ones_v):
    cid = lax.axis_index("c")
    sid = lax.axis_index("s")
    wid = cid * NS + sid
    row0 = sid * RPT
    pltpu.sync_copy(zeros_hbm, deg_sh.at[pl.ds(row0, RPT)])
    pltpu.sync_copy(ones_hbm, ones_v)
    pltpu.sync_copy(dst_hbm.at[wid], dst_v)
    plsc.subcore_barrier()

    def chunk(j, _):
        pltpu.sync_copy(ones_v, deg_sh.at[dst_v.at[j]], add=True)
        return _

    lax.fori_loop(0, NCHUNK, chunk, None)
    plsc.subcore_barrier()
    pltpu.sync_copy(deg_sh.at[pl.ds(row0, RPT)], out_hbm.at[cid, pl.ds(row0, RPT)])


def _sc_degree(dst_r, ones128, zeros128):
    f = pl.kernel(
        _deg_body,
        out_type=jax.ShapeDtypeStruct((NC, NP, H), jnp.float32),
        mesh=_sc_mesh(),
        scratch_types=[
            pltpu.VMEM_SHARED((NP, H), jnp.float32),
            pltpu.VMEM((NCHUNK, CS), jnp.int32),
            pltpu.VMEM((CS, H), jnp.float32),
        ],
    )
    return f(dst_r, ones128, zeros128)


def _acc_body(h_hbm, src_hbm, dst_hbm, zeros_hbm, out_hbm,
              acc_sh, src_v, dst_v, rows_v, sem):
    cid = lax.axis_index("c")
    sid = lax.axis_index("s")
    wid = cid * NS + sid
    row0 = sid * RPT
    pltpu.sync_copy(zeros_hbm, acc_sh.at[pl.ds(row0, RPT)])
    pltpu.sync_copy(src_hbm.at[wid], src_v)
    pltpu.sync_copy(dst_hbm.at[wid], dst_v)
    plsc.subcore_barrier()

    def chunk(j, _):
        pltpu.async_copy(h_hbm.at[src_v.at[j]], rows_v, sem).wait()
        pltpu.sync_copy(rows_v, acc_sh.at[dst_v.at[j]], add=True)
        return _

    lax.fori_loop(0, NCHUNK, chunk, None)
    plsc.subcore_barrier()
    pltpu.sync_copy(acc_sh.at[pl.ds(row0, RPT)], out_hbm.at[cid, pl.ds(row0, RPT)])


def _sc_accumulate(h_s, src_r, dst_r, zeros128):
    f = pl.kernel(
        _acc_body,
        out_type=jax.ShapeDtypeStruct((NC, NP, H), jnp.float32),
        mesh=_sc_mesh(),
        scratch_types=[
            pltpu.VMEM_SHARED((NP, H), jnp.float32),
            pltpu.VMEM((NCHUNK, CS), jnp.int32),
            pltpu.VMEM((NCHUNK, CS), jnp.int32),
            pltpu.VMEM((CS, H), jnp.float32),
            pltpu.SemaphoreType.DMA,
        ],
    )
    return f(h_s, src_r, dst_r, zeros128)


R = 400


def _dinv(deg0, deg1):
    return lax.rsqrt(deg0[:, 0:1] + deg1[:, 0:1] + 1.0)


def _first_body(x_ref, w_ref, deg0_ref, deg1_ref, o_ref):
    dinv = _dinv(deg0_ref[...], deg1_ref[...])
    h = jnp.dot(x_ref[...], w_ref[...], preferred_element_type=jnp.float32)
    o_ref[...] = h * dinv


def _mid_body(a0_ref, a1_ref, hs_ref, deg0_ref, deg1_ref, b_ref, w_ref, o_ref):
    dinv = _dinv(deg0_ref[...], deg1_ref[...])
    z = a0_ref[...] + a1_ref[...] + hs_ref[...]
    z = jnp.maximum(z * dinv + b_ref[...], 0.0)
    o_ref[...] = jnp.dot(z, w_ref[...], preferred_element_type=jnp.float32) * dinv


def _head_body(a0_ref, a1_ref, hs_ref, deg0_ref, deg1_ref, b3_ref,
               wl1_ref, bl1_ref, wl2_ref, bl2_ref, o_ref):
    dinv = _dinv(deg0_ref[...], deg1_ref[...])
    z = a0_ref[...] + a1_ref[...] + hs_ref[...]
    z = jnp.maximum(z * dinv + b3_ref[...], 0.0)
    h4 = jnp.dot(z, wl1_ref[...], preferred_element_type=jnp.float32)
    h4 = jnp.maximum(h4 + bl1_ref[...], 0.0)
    logits = jnp.dot(h4, wl2_ref[...], preferred_element_type=jnp.float32)
    logits = logits + bl2_ref[...]
    col = lax.broadcasted_iota(jnp.int32, logits.shape, 1)
    logits = jnp.where(col < C, logits, -1e30)
    m = jnp.max(logits, axis=-1, keepdims=True)
    lse = jnp.log(jnp.sum(jnp.exp(logits - m), axis=-1, keepdims=True))
    o_ref[...] = logits - m - lse


def _row_spec(width):
    return pl.BlockSpec((R, width), lambda i: (i, 0))


def _full_spec(shape):
    return pl.BlockSpec(shape, lambda i: (0,) * len(shape))


def _tc_first(x, w, deg0, deg1):
    return pl.pallas_call(
        _first_body,
        grid=(N // R,),
        in_specs=[_row_spec(H), _full_spec((H, H)), _row_spec(16), _row_spec(16)],
        out_specs=_row_spec(H),
        out_shape=jax.ShapeDtypeStruct((N, H), jnp.float32),
    )(x, w, deg0, deg1)


def _tc_mid(a0, a1, hs, deg0, deg1, b, w):
    return pl.pallas_call(
        _mid_body,
        grid=(N // R,),
        in_specs=[_row_spec(H), _row_spec(H), _row_spec(H), _row_spec(16),
                  _row_spec(16), _full_spec((1, H)), _full_spec((H, H))],
        out_specs=_row_spec(H),
        out_shape=jax.ShapeDtypeStruct((N, H), jnp.float32),
    )(a0, a1, hs, deg0, deg1, b, w)


def _tc_head(a0, a1, hs, deg0, deg1, b3, wl1, bl1, wl2p, bl2p):
    return pl.pallas_call(
        _head_body,
        grid=(N // R,),
        in_specs=[_row_spec(H), _row_spec(H), _row_spec(H), _row_spec(16),
                  _row_spec(16), _full_spec((1, H)), _full_spec((H, H)),
                  _full_spec((1, H)), _full_spec((H, H)), _full_spec((1, H))],
        out_specs=_row_spec(H),
        out_shape=jax.ShapeDtypeStruct((N, H), jnp.float32),
    )(a0, a1, hs, deg0, deg1, b3, wl1, bl1, wl2p, bl2p)


def kernel(x, edge_index, batch, W1, b1, W2, b2, W3, b3, Wl1, bl1, Wl2, bl2):
    del batch
    src_r = edge_index[0].reshape(NW, NCHUNK, CS)
    dst_r = edge_index[1].reshape(NW, NCHUNK, CS)
    ones128 = jnp.ones((CS, H), jnp.float32)
    zeros128 = jnp.zeros((RPT, H), jnp.float32)

    deg = _sc_degree(dst_r, ones128, zeros128)
    deg0, deg1 = deg[0, :N, :16], deg[1, :N, :16]

    hs = _tc_first(x, W1, deg0, deg1)
    acc = _sc_accumulate(hs, src_r, dst_r, zeros128)
    hs = _tc_mid(acc[0, :N], acc[1, :N], hs, deg0, deg1, b1.reshape(1, H), W2)
    acc = _sc_accumulate(hs, src_r, dst_r, zeros128)
    hs = _tc_mid(acc[0, :N], acc[1, :N], hs, deg0, deg1, b2.reshape(1, H), W3)
    acc = _sc_accumulate(hs, src_r, dst_r, zeros128)

    wl2p = jnp.pad(Wl2, ((0, 0), (0, H - C)))
    bl2p = jnp.pad(bl2, (0, H - C)).reshape(1, H)
    out = _tc_head(acc[0, :N], acc[1, :N], hs, deg0, deg1, b3.reshape(1, H),
                   Wl1, bl1.reshape(1, H), wl2p, bl2p)
    return out[:, :C]
